# Initial kernel scaffold; baseline (speedup 1.0000x reference)
import jax
import jax.numpy as jnp
from jax.experimental import pallas as pl


def kernel(x, edge_index, edge_attr, params, r):
    raise NotImplementedError("write your pallas kernel here")


# speedup vs baseline: 2.1134x; 2.1134x over previous
import functools

import jax
import jax.numpy as jnp
from jax import lax
from jax.experimental import pallas as pl
from jax.experimental.pallas import tpu as pltpu
from jax.experimental.pallas import tpu_sc as plsc

_NC = 2
_NS = 16
_LANES = 16


def _enc_body(n_nodes, hh, x_ref, w_ref, o_ref):
    r = jnp.dot(x_ref[...], w_ref[...], preferred_element_type=jnp.float32)
    o_ref[0:n_nodes, :] = r[:, 0:hh]
    o_ref[n_nodes:, :] = r[:, hh:]


def _encode(x, w):
    n, _ = x.shape
    h_dim = w.shape[1]
    hh = h_dim // 2
    return pl.pallas_call(
        functools.partial(_enc_body, n, hh),
        out_shape=jax.ShapeDtypeStruct((2 * n, hh), jnp.float32),
    )(x, w)


def _edge_body(hh, a_ref, w_ref, o_ref):
    r = jnp.dot(a_ref[...], w_ref[...], preferred_element_type=jnp.float32)
    o_ref[0] = r[:, 0:hh]
    o_ref[1] = r[:, hh:]


def _edge_mm(edge_attr, w_e, block_rows=3200):
    e_cnt, de = edge_attr.shape
    h_dim = w_e.shape[1]
    hh = h_dim // 2
    grid = e_cnt // block_rows
    return pl.pallas_call(
        functools.partial(_edge_body, hh),
        grid=(grid,),
        in_specs=[
            pl.BlockSpec((block_rows, de), lambda i: (i, 0)),
            pl.BlockSpec((de, h_dim), lambda i: (0, 0)),
        ],
        out_specs=pl.BlockSpec((2, block_rows, hh), lambda i: (0, i, 0)),
        out_shape=jax.ShapeDtypeStruct((2, e_cnt, hh), jnp.float32),
    )(edge_attr, w_e)


def _layer_body(n_nodes, hh, h_ref, a_ref, s_ref, w1_ref, b1_ref, w2_ref,
                b2_ref, o_ref):
    hf = jnp.concatenate([h_ref[0:n_nodes, :], h_ref[n_nodes:, :]], axis=1)
    af = jnp.concatenate([a_ref[0:n_nodes, :], a_ref[n_nodes:, :]], axis=1)
    t = hf * s_ref[...] + af
    z = jnp.maximum(
        jnp.dot(t, w1_ref[...], preferred_element_type=jnp.float32)
        + b1_ref[...], 0.0)
    hn = jnp.maximum(
        jnp.dot(z, w2_ref[...], preferred_element_type=jnp.float32)
        + b2_ref[...], 0.0)
    o_ref[0:n_nodes, :] = hn[:, 0:hh]
    o_ref[n_nodes:, :] = hn[:, hh:]


def _layer(h2, agg2, scale_row, w1, b1, w2, b2):
    n2, hh = h2.shape
    n = n2 // 2
    return pl.pallas_call(
        functools.partial(_layer_body, n, hh),
        out_shape=jax.ShapeDtypeStruct((n2, hh), jnp.float32),
    )(h2, agg2, scale_row, w1, b1, w2, b2)


def _proj_body(n_nodes, h_ref, w_ref, b_ref, o_ref):
    hf = jnp.concatenate([h_ref[0:n_nodes, :], h_ref[n_nodes:, :]], axis=1)
    o_ref[...] = jnp.dot(hf, w_ref[...],
                         preferred_element_type=jnp.float32) + b_ref[...]


def _proj(h2, w_out, b_row):
    n = h2.shape[0] // 2
    out = w_out.shape[1]
    return pl.pallas_call(
        functools.partial(_proj_body, n),
        out_shape=jax.ShapeDtypeStruct((n, out), jnp.float32),
    )(h2, w_out, b_row)


def _make_sc_aggregate(n_nodes, hh, n_chunks, chunk):
    slab_rows = (n_nodes // _NS) & ~7
    tail_rows = n_nodes - _NS * slab_rows
    mesh = plsc.VectorSubcoreMesh(core_axis_name="c", subcore_axis_name="s")

    @functools.partial(
        pl.kernel,
        mesh=mesh,
        out_type=jax.ShapeDtypeStruct((_NC * n_nodes, hh), jnp.float32),
        scratch_types=[
            pltpu.VMEM((n_chunks, chunk), jnp.int32),
            pltpu.VMEM((n_chunks, chunk), jnp.int32),
            pltpu.VMEM((chunk, hh), jnp.float32),
            pltpu.VMEM((chunk, hh), jnp.float32),
            pltpu.VMEM((chunk, hh), jnp.float32),
            pltpu.VMEM_SHARED((n_nodes, hh), jnp.float32),
            pltpu.SemaphoreType.DMA,
        ],
        compiler_params=pltpu.CompilerParams(use_tc_tiling_on_sc=False),
    )
    def sc_aggregate(h_hbm, e_hbm, src_hbm, dst_hbm, z_hbm, out_hbm,
                     src_v, dst_v, gath_v, e_v, msg_v, agg_sh, sem):
        cid = lax.axis_index("c")
        sid = lax.axis_index("s")
        slab = pl.ds(sid * slab_rows, slab_rows)
        tail = pl.ds(_NS * slab_rows, tail_rows)
        pltpu.sync_copy(z_hbm.at[slab], agg_sh.at[slab])

        @pl.when(sid == _NS - 1)
        def _():
            pltpu.sync_copy(z_hbm.at[tail], agg_sh.at[tail])

        pltpu.sync_copy(src_hbm.at[sid], src_v)
        pltpu.sync_copy(dst_hbm.at[sid], dst_v)
        row_off = cid * n_nodes

        def offs(i, c2):
            for k in range(chunk // _LANES):
                sl = pl.ds(k * _LANES, _LANES)
                src_v[i, sl] = src_v[i, sl] + row_off
            return c2

        lax.fori_loop(0, n_chunks, offs, 0)
        plsc.subcore_barrier()

        chunk_base = (cid * _NS + sid) * n_chunks

        def do_chunk(j, carry):
            pltpu.sync_copy(e_hbm.at[chunk_base + j], e_v)
            pltpu.async_copy(h_hbm.at[src_v.at[j]], gath_v, sem).wait()

            def row(i, c2):
                for k in range(hh // _LANES):
                    sl = pl.ds(k * _LANES, _LANES)
                    msg_v[i, sl] = jnp.maximum(gath_v[i, sl] + e_v[i, sl],
                                               0.0)
                return c2

            lax.fori_loop(0, chunk, row, 0)
            pltpu.sync_copy(msg_v, agg_sh.at[dst_v.at[j]], add=True)
            return carry

        lax.fori_loop(0, n_chunks, do_chunk, 0)
        plsc.subcore_barrier()
        pltpu.sync_copy(agg_sh.at[slab],
                        out_hbm.at[pl.ds(cid * n_nodes + sid * slab_rows,
                                         slab_rows)])

        @pl.when(sid == _NS - 1)
        def _():
            pltpu.sync_copy(
                agg_sh.at[tail],
                out_hbm.at[pl.ds(cid * n_nodes + _NS * slab_rows,
                                 tail_rows)])

    return sc_aggregate


def kernel(x, edge_index, edge_attr, params, r):
    n_nodes, _ = x.shape
    e_cnt = edge_index.shape[1]
    h_dim = params['W_enc'].shape[1]
    hh = h_dim // 2

    ept = e_cnt // _NS
    chunk = 80
    n_chunks = ept // chunk

    h2 = _encode(x, params['W_enc'])
    e2 = _edge_mm(edge_attr, params['W_e'])

    src = edge_index[0].reshape(_NS, n_chunks, chunk)
    dst = edge_index[1].reshape(_NS, n_chunks, chunk)
    e3 = e2.reshape(_NC * _NS * n_chunks, chunk, hh)
    zeros = jnp.zeros((n_nodes, hh), jnp.float32)

    sc_aggregate = _make_sc_aggregate(n_nodes, hh, n_chunks, chunk)

    ones_row = jnp.ones((1, h_dim), jnp.float32)
    for lyr in params['layers']:
        agg2 = sc_aggregate(h2, e3, src, dst, zeros)
        scale_row = ones_row * (1.0 + lyr['eps'])
        h2 = _layer(h2, agg2, scale_row, lyr['W1'],
                    lyr['b1'].reshape(1, h_dim), lyr['W2'],
                    lyr['b2'].reshape(1, h_dim))

    out_dim = params['W_out'].shape[1]
    return _proj(h2, params['W_out'], params['b_out'].reshape(1, out_dim))


# speedup vs baseline: 4.0698x; 1.9257x over previous
import functools

import jax
import jax.numpy as jnp
from jax import lax
from jax.experimental import pallas as pl
from jax.experimental.pallas import tpu as pltpu
from jax.experimental.pallas import tpu_sc as plsc

_NC = 2
_NS = 16
_LANES = 16


def _enc_body(n_nodes, hh, x_ref, w_ref, o_ref):
    r = jnp.dot(x_ref[...], w_ref[...], preferred_element_type=jnp.float32)
    o_ref[0:n_nodes, :] = r[:, 0:hh]
    o_ref[n_nodes:, :] = r[:, hh:]


def _encode(x, w):
    n, _ = x.shape
    h_dim = w.shape[1]
    hh = h_dim // 2
    return pl.pallas_call(
        functools.partial(_enc_body, n, hh),
        out_shape=jax.ShapeDtypeStruct((2 * n, hh), jnp.float32),
    )(x, w)


def _edge_body(hh, a_ref, w_ref, o_ref):
    r = jnp.dot(a_ref[...], w_ref[...], preferred_element_type=jnp.float32)
    o_ref[0] = r[:, 0:hh]
    o_ref[1] = r[:, hh:]


def _edge_mm(edge_attr, w_e, block_rows=3200):
    e_cnt, de = edge_attr.shape
    h_dim = w_e.shape[1]
    hh = h_dim // 2
    grid = e_cnt // block_rows
    return pl.pallas_call(
        functools.partial(_edge_body, hh),
        grid=(grid,),
        in_specs=[
            pl.BlockSpec((block_rows, de), lambda i: (i, 0)),
            pl.BlockSpec((de, h_dim), lambda i: (0, 0)),
        ],
        out_specs=pl.BlockSpec((2, block_rows, hh), lambda i: (0, i, 0)),
        out_shape=jax.ShapeDtypeStruct((2, e_cnt, hh), jnp.float32),
    )(edge_attr, w_e)


def _layer_body(n_nodes, hh, h_ref, a_ref, s_ref, w1_ref, b1_ref, w2_ref,
                b2_ref, o_ref):
    hf = jnp.concatenate([h_ref[0:n_nodes, :], h_ref[n_nodes:, :]], axis=1)
    af = jnp.concatenate([a_ref[0:n_nodes, :], a_ref[n_nodes:, :]], axis=1)
    t = hf * s_ref[...] + af
    z = jnp.maximum(
        jnp.dot(t, w1_ref[...], preferred_element_type=jnp.float32)
        + b1_ref[...], 0.0)
    hn = jnp.maximum(
        jnp.dot(z, w2_ref[...], preferred_element_type=jnp.float32)
        + b2_ref[...], 0.0)
    o_ref[0:n_nodes, :] = hn[:, 0:hh]
    o_ref[n_nodes:, :] = hn[:, hh:]


def _layer(h2, agg2, scale_row, w1, b1, w2, b2):
    n2, hh = h2.shape
    n = n2 // 2
    return pl.pallas_call(
        functools.partial(_layer_body, n, hh),
        out_shape=jax.ShapeDtypeStruct((n2, hh), jnp.float32),
    )(h2, agg2, scale_row, w1, b1, w2, b2)


def _proj_body(n_nodes, h_ref, w_ref, b_ref, o_ref):
    hf = jnp.concatenate([h_ref[0:n_nodes, :], h_ref[n_nodes:, :]], axis=1)
    o_ref[...] = jnp.dot(hf, w_ref[...],
                         preferred_element_type=jnp.float32) + b_ref[...]


def _proj(h2, w_out, b_row):
    n = h2.shape[0] // 2
    out = w_out.shape[1]
    return pl.pallas_call(
        functools.partial(_proj_body, n),
        out_shape=jax.ShapeDtypeStruct((n, out), jnp.float32),
    )(h2, w_out, b_row)


def _make_sc_aggregate(n_nodes, hh, n_chunks, chunk):
    slab_rows = (n_nodes // _NS) & ~7
    tail_rows = n_nodes - _NS * slab_rows
    mesh = plsc.VectorSubcoreMesh(core_axis_name="c", subcore_axis_name="s")

    @functools.partial(
        pl.kernel,
        mesh=mesh,
        out_type=jax.ShapeDtypeStruct((_NC * n_nodes, hh), jnp.float32),
        scratch_types=[
            pltpu.VMEM((n_chunks, chunk), jnp.int32),
            pltpu.VMEM((n_chunks, chunk), jnp.int32),
            pltpu.VMEM((2, chunk, hh), jnp.float32),
            pltpu.VMEM((2, chunk, hh), jnp.float32),
            pltpu.VMEM((2, chunk, hh), jnp.float32),
            pltpu.VMEM_SHARED((n_nodes, hh), jnp.float32),
            pltpu.SemaphoreType.DMA,
            pltpu.SemaphoreType.DMA,
            pltpu.SemaphoreType.DMA,
            pltpu.SemaphoreType.DMA,
            pltpu.SemaphoreType.DMA,
            pltpu.SemaphoreType.DMA,
        ],
        compiler_params=pltpu.CompilerParams(use_tc_tiling_on_sc=False),
    )
    def sc_aggregate(h_hbm, e_hbm, src_hbm, dst_hbm, z_hbm, out_hbm,
                     src_v, dst_v, gath_v, e_v, msg_v, agg_sh,
                     gsem0, gsem1, esem0, esem1, ssem0, ssem1):
        cid = lax.axis_index("c")
        sid = lax.axis_index("s")
        slab = pl.ds(sid * slab_rows, slab_rows)
        tail = pl.ds(_NS * slab_rows, tail_rows)
        pltpu.sync_copy(z_hbm.at[slab], agg_sh.at[slab])

        @pl.when(sid == _NS - 1)
        def _():
            pltpu.sync_copy(z_hbm.at[tail], agg_sh.at[tail])

        pltpu.sync_copy(src_hbm.at[sid], src_v)
        pltpu.sync_copy(dst_hbm.at[sid], dst_v)
        row_off = cid * n_nodes

        def offs(i, c2):
            for k in range(chunk // _LANES):
                sl = pl.ds(k * _LANES, _LANES)
                src_v[i, sl] = src_v[i, sl] + row_off
            return c2

        lax.fori_loop(0, n_chunks, offs, 0)
        plsc.subcore_barrier()

        chunk_base = (cid * _NS + sid) * n_chunks
        gsems = (gsem0, gsem1)
        esems = (esem0, esem1)
        ssems = (ssem0, ssem1)

        def issue(j, b):
            pltpu.async_copy(e_hbm.at[chunk_base + j], e_v.at[b], esems[b])
            pltpu.async_copy(h_hbm.at[src_v.at[j]], gath_v.at[b], gsems[b])

        def wait_in(j, b):
            pltpu.make_async_copy(e_hbm.at[chunk_base + j], e_v.at[b],
                                  esems[b]).wait()
            pltpu.make_async_copy(h_hbm.at[src_v.at[j]], gath_v.at[b],
                                  gsems[b]).wait()

        def compute(b):
            def rows(i4, c2):
                for r4 in range(4):
                    for k in range(hh // _LANES):
                        sl = pl.ds(k * _LANES, _LANES)
                        i = i4 * 4 + r4
                        msg_v[b, i, sl] = jnp.maximum(
                            gath_v[b, i, sl] + e_v[b, i, sl], 0.0)
                return c2

            lax.fori_loop(0, chunk // 4, rows, 0)

        issue(0, 0)

        def do_pair(jj, carry):
            for b in range(2):
                j = jj * 2 + b

                @pl.when(j + 1 < n_chunks)
                def _():
                    issue(j + 1, 1 - b)

                wait_in(j, b)

                @pl.when(j >= 2)
                def _():
                    pltpu.make_async_copy(msg_v.at[b],
                                          agg_sh.at[dst_v.at[j - 2]],
                                          ssems[b]).wait()

                compute(b)
                pltpu.async_copy(msg_v.at[b], agg_sh.at[dst_v.at[j]],
                                 ssems[b], add=True)
            return carry

        lax.fori_loop(0, n_chunks // 2, do_pair, 0)
        for b in range(2):
            pltpu.make_async_copy(msg_v.at[b],
                                  agg_sh.at[dst_v.at[n_chunks - 2 + b]],
                                  ssems[b]).wait()
        plsc.subcore_barrier()
        pltpu.sync_copy(agg_sh.at[slab],
                        out_hbm.at[pl.ds(cid * n_nodes + sid * slab_rows,
                                         slab_rows)])

        @pl.when(sid == _NS - 1)
        def _():
            pltpu.sync_copy(
                agg_sh.at[tail],
                out_hbm.at[pl.ds(cid * n_nodes + _NS * slab_rows,
                                 tail_rows)])

    return sc_aggregate


def kernel(x, edge_index, edge_attr, params, r):
    n_nodes, _ = x.shape
    e_cnt = edge_index.shape[1]
    h_dim = params['W_enc'].shape[1]
    hh = h_dim // 2

    ept = e_cnt // _NS
    chunk = 80
    n_chunks = ept // chunk

    h2 = _encode(x, params['W_enc'])
    e2 = _edge_mm(edge_attr, params['W_e'])

    src = edge_index[0].reshape(_NS, n_chunks, chunk)
    dst = edge_index[1].reshape(_NS, n_chunks, chunk)
    e3 = e2.reshape(_NC * _NS * n_chunks, chunk, hh)
    zeros = jnp.zeros((n_nodes, hh), jnp.float32)

    sc_aggregate = _make_sc_aggregate(n_nodes, hh, n_chunks, chunk)

    ones_row = jnp.ones((1, h_dim), jnp.float32)
    for lyr in params['layers']:
        agg2 = sc_aggregate(h2, e3, src, dst, zeros)
        scale_row = ones_row * (1.0 + lyr['eps'])
        h2 = _layer(h2, agg2, scale_row, lyr['W1'],
                    lyr['b1'].reshape(1, h_dim), lyr['W2'],
                    lyr['b2'].reshape(1, h_dim))

    out_dim = params['W_out'].shape[1]
    return _proj(h2, params['W_out'], params['b_out'].reshape(1, out_dim))
